# baseline (speedup 1.0000x reference)
import functools

import jax
import jax.numpy as jnp
from jax import lax
from jax.experimental import pallas as pl
from jax.experimental.pallas import tpu as pltpu
from jax.experimental.pallas import tpu_sc as plsc

HIDDEN = 768
N_EXPERTS = 64
TOP_K = 8
N_GROUP = 8
TOPK_GROUP = 4
GROUP_SIZE = N_EXPERTS // N_GROUP
AUX_ALPHA = 0.001
Z_ALPHA = 0.0001
TOKENS = 32768

TILE_T = 2048
NSPLIT = 4
LROW = 128

NC, NS = 2, 16
NW = NC * NS
CH = 256


def _gate_body(x_ref, wt_ref, logits_ref, s_ref):
    i = pl.program_id(0)
    x = x_ref[...]
    w = wt_ref[...]
    logits = lax.dot_general(x, w, (((1,), (1,)), ((), ())),
                             preferred_element_type=jnp.float32,
                             precision=lax.Precision.DEFAULT)
    logits_ref[...] = jnp.concatenate(
        [logits, jnp.zeros((TILE_T, LROW - N_EXPERTS), jnp.float32)], axis=1)
    m = jnp.max(logits, axis=1, keepdims=True)
    p = jnp.exp(logits - m)
    s = jnp.sum(p, axis=1, keepdims=True)
    colsum = jnp.sum(p / s, axis=0)

    @pl.when(i == 0)
    def _():
        s_ref[...] = jnp.zeros_like(s_ref)

    s_ref[...] += colsum


def _gate_call(hidden_states, wt, off, n):
    return pl.pallas_call(
        _gate_body,
        grid=(n,),
        in_specs=[
            pl.BlockSpec((TILE_T, HIDDEN), lambda i, off=off: (i + off, 0)),
            pl.BlockSpec((N_EXPERTS, HIDDEN), lambda i: (0, 0)),
        ],
        out_specs=[
            pl.BlockSpec((TILE_T, LROW), lambda i: (i, 0)),
            pl.BlockSpec((N_EXPERTS,), lambda i: (0,)),
        ],
        out_shape=[
            jax.ShapeDtypeStruct((n * TILE_T, LROW), jnp.float32),
            jax.ShapeDtypeStruct((N_EXPERTS,), jnp.float32),
        ],
    )(hidden_states, wt)


def _vgather(v, idx):
    return v.at[idx].get(mode="promise_in_bounds")


def _make_sc_body(tw):
    def _sc_topk_body(logits_hbm, idx_hbm, w_hbm, cnt_hbm,
                      lbuf, oi, ow, cnt64):
        wid = lax.axis_index("c") * NS + lax.axis_index("s")
        base = wid * tw
        iota = lax.iota(jnp.int32, 16)
        lt8 = iota < 8
        rot8 = (iota + 8) & 15
        ones8 = jnp.where(lt8, 1.0, 0.0)

        for j in range(4):
            cnt64[pl.ds(16 * j, 16)] = jnp.zeros((16,), jnp.float32)

        for c in range(tw // CH):
            tok0 = base + c * CH
            pltpu.sync_copy(
                logits_hbm.at[pl.ds(tok0 * LROW, CH * LROW)], lbuf)

            @plsc.parallel_loop(0, CH, unroll=2)
            def _token(t):
                packed = []
                for q in range(4):
                    x = lbuf[pl.ds(t * LROW + 16 * q, 16)]
                    idxq = iota + 16 * q
                    kd, vd = plsc.sort_key_val(x, idxq, descending=True)
                    bit = (vd >> 3) & 1
                    cs = plsc.cumsum(bit)
                    rank = jnp.where(bit == 1, cs - 1, iota - cs)
                    keep = rank < TOPK_GROUP
                    ck, cv = plsc.sort_key_val(
                        jnp.where(keep, kd, -1e30), vd, descending=True)
                    packed.append((ck, cv))

                def merge(a, b):
                    return jnp.where(lt8, a, _vgather(b, rot8))

                mk1, mv1 = plsc.sort_key_val(
                    merge(packed[0][0], packed[1][0]),
                    merge(packed[0][1], packed[1][1]), descending=True)
                mk2, mv2 = plsc.sort_key_val(
                    merge(packed[2][0], packed[3][0]),
                    merge(packed[2][1], packed[3][1]), descending=True)
                kk, vv = plsc.sort_key_val(merge(mk1, mk2), merge(mv1, mv2),
                                           descending=True)
                m = jnp.max(kk)
                e = jnp.where(lt8, jnp.exp(kk - m), 0.0)
                s = jnp.sum(e)
                w = e / jnp.broadcast_to(s, (16,))
                plsc.store_compressed(oi.at[pl.ds(t * TOP_K, 16)], vv,
                                      mask=lt8)
                plsc.store_compressed(ow.at[pl.ds(t * TOP_K, 16)], w,
                                      mask=lt8)
                plsc.addupdate_scatter(cnt64, [vv], ones8, mask=lt8)

            pltpu.sync_copy(oi.at[pl.ds(0, CH * TOP_K)],
                            idx_hbm.at[pl.ds(tok0 * TOP_K, CH * TOP_K)])
            pltpu.sync_copy(ow.at[pl.ds(0, CH * TOP_K)],
                            w_hbm.at[pl.ds(tok0 * TOP_K, CH * TOP_K)])

        pltpu.sync_copy(cnt64, cnt_hbm.at[pl.ds(wid * N_EXPERTS, N_EXPERTS)])

    return _sc_topk_body


def _sc_call(logits, tokens):
    tw = tokens // NW
    mesh = plsc.VectorSubcoreMesh(core_axis_name="c", subcore_axis_name="s")
    f = functools.partial(
        pl.kernel, mesh=mesh,
        out_type=[
            jax.ShapeDtypeStruct((tokens * TOP_K,), jnp.int32),
            jax.ShapeDtypeStruct((tokens * TOP_K,), jnp.float32),
            jax.ShapeDtypeStruct((NW * N_EXPERTS,), jnp.float32),
        ],
        scratch_types=[
            pltpu.VMEM((CH * LROW,), jnp.float32),
            pltpu.VMEM((CH * TOP_K + 8,), jnp.int32),
            pltpu.VMEM((CH * TOP_K + 8,), jnp.float32),
            pltpu.VMEM((N_EXPERTS,), jnp.float32),
        ],
        compiler_params=pltpu.CompilerParams(needs_layout_passes=False),
    )(_make_sc_body(tw))
    return f(logits.reshape(-1))


def _loss_body(*refs):
    cnt_refs = refs[:NSPLIT]
    s_refs = refs[NSPLIT:2 * NSPLIT]
    out_ref = refs[2 * NSPLIT]
    cnt = jnp.zeros((N_EXPERTS,), jnp.float32)
    for r in cnt_refs:
        for j in range(NW):
            cnt = cnt + r[pl.ds(j * N_EXPERTS, N_EXPERTS)]
    s = s_refs[0][...]
    for r in s_refs[1:]:
        s = s + r[...]
    aux = jnp.sum(cnt * s) * (AUX_ALPHA / (TOKENS * float(TOKENS)))
    z = jnp.mean(jnp.log(s) ** 2) * Z_ALPHA
    out_ref[...] = jnp.broadcast_to(aux + z, (1, 1))


def _loss_call(cnts, ss):
    return pl.pallas_call(
        _loss_body,
        out_shape=jax.ShapeDtypeStruct((1, 1), jnp.float32),
    )(*cnts, *ss)


def kernel(hidden_states, gate_weight):
    wt = gate_weight
    tiles = TOKENS // TILE_T
    per = tiles // NSPLIT
    gates = [_gate_call(hidden_states, wt, j * per, per)
             for j in range(NSPLIT)]
    scs = [_sc_call(lg, per * TILE_T) for (lg, _) in gates]
    topk_idx = jnp.concatenate([o[0] for o in scs]).reshape(TOKENS, TOP_K)
    topk_weight = jnp.concatenate([o[1] for o in scs]).reshape(TOKENS, TOP_K)
    loss = _loss_call([o[2] for o in scs], [g[1] for g in gates])
    return (topk_idx, topk_weight, loss[0, 0])


# speedup vs baseline: 1.0162x; 1.0162x over previous
import functools

import jax
import jax.numpy as jnp
from jax import lax
from jax.experimental import pallas as pl
from jax.experimental.pallas import tpu as pltpu
from jax.experimental.pallas import tpu_sc as plsc

HIDDEN = 768
N_EXPERTS = 64
TOP_K = 8
N_GROUP = 8
TOPK_GROUP = 4
GROUP_SIZE = N_EXPERTS // N_GROUP
AUX_ALPHA = 0.001
Z_ALPHA = 0.0001
TOKENS = 32768

TILE_T = 2048
NSPLIT = 4
LROW = 128

NC, NS = 2, 16
NW = NC * NS
CH = 128


def _gate_body(x_ref, wt_ref, logits_ref, s_ref):
    i = pl.program_id(0)
    x = x_ref[...]
    w = wt_ref[...]
    logits = lax.dot_general(x, w, (((1,), (1,)), ((), ())),
                             preferred_element_type=jnp.float32,
                             precision=lax.Precision.DEFAULT)
    logits_ref[...] = jnp.concatenate(
        [logits, jnp.zeros((TILE_T, LROW - N_EXPERTS), jnp.float32)], axis=1)
    m = jnp.max(logits, axis=1, keepdims=True)
    p = jnp.exp(logits - m)
    s = jnp.sum(p, axis=1, keepdims=True)
    colsum = jnp.sum(p / s, axis=0)

    @pl.when(i == 0)
    def _():
        s_ref[...] = jnp.zeros_like(s_ref)

    s_ref[...] += colsum


def _gate_call(hidden_states, wt, off, n):
    return pl.pallas_call(
        _gate_body,
        grid=(n,),
        in_specs=[
            pl.BlockSpec((TILE_T, HIDDEN), lambda i, off=off: (i + off, 0)),
            pl.BlockSpec((N_EXPERTS, HIDDEN), lambda i: (0, 0)),
        ],
        out_specs=[
            pl.BlockSpec((TILE_T, LROW), lambda i: (i, 0)),
            pl.BlockSpec((N_EXPERTS,), lambda i: (0,)),
        ],
        out_shape=[
            jax.ShapeDtypeStruct((n * TILE_T, LROW), jnp.float32),
            jax.ShapeDtypeStruct((N_EXPERTS,), jnp.float32),
        ],
    )(hidden_states, wt)


def _vgather(v, idx):
    return v.at[idx].get(mode="promise_in_bounds")


def _make_sc_body(tw):
    def _sc_topk_body(logits_hbm, idx_hbm, w_hbm, cnt_hbm,
                      lbuf, oi, ow, cnt64):
        wid = lax.axis_index("c") * NS + lax.axis_index("s")
        base = wid * tw
        iota = lax.iota(jnp.int32, 16)
        lt8 = iota < 8
        rot8 = (iota + 8) & 15
        ones8 = jnp.where(lt8, 1.0, 0.0)

        for j in range(4):
            cnt64[pl.ds(16 * j, 16)] = jnp.zeros((16,), jnp.float32)

        for c in range(tw // CH):
            tok0 = base + c * CH
            pltpu.sync_copy(
                logits_hbm.at[pl.ds(tok0 * LROW, CH * LROW)], lbuf)

            @plsc.parallel_loop(0, CH, unroll=2)
            def _token(t):
                packed = []
                for q in range(4):
                    x = lbuf[pl.ds(t * LROW + 16 * q, 16)]
                    idxq = iota + 16 * q
                    kd, vd = plsc.sort_key_val(x, idxq, descending=True)
                    bit = (vd >> 3) & 1
                    cs = plsc.cumsum(bit)
                    rank = jnp.where(bit == 1, cs - 1, iota - cs)
                    keep = rank < TOPK_GROUP
                    ck, cv = plsc.sort_key_val(
                        jnp.where(keep, kd, -1e30), vd, descending=True)
                    packed.append((ck, cv))

                def merge(a, b):
                    return jnp.where(lt8, a, _vgather(b, rot8))

                mk1, mv1 = plsc.sort_key_val(
                    merge(packed[0][0], packed[1][0]),
                    merge(packed[0][1], packed[1][1]), descending=True)
                mk2, mv2 = plsc.sort_key_val(
                    merge(packed[2][0], packed[3][0]),
                    merge(packed[2][1], packed[3][1]), descending=True)
                kk, vv = plsc.sort_key_val(merge(mk1, mk2), merge(mv1, mv2),
                                           descending=True)
                m = jnp.max(kk)
                e = jnp.where(lt8, jnp.exp(kk - m), 0.0)
                s = jnp.sum(e)
                w = e / jnp.broadcast_to(s, (16,))
                plsc.store_compressed(oi.at[pl.ds(t * TOP_K, 16)], vv,
                                      mask=lt8)
                plsc.store_compressed(ow.at[pl.ds(t * TOP_K, 16)], w,
                                      mask=lt8)
                plsc.addupdate_scatter(cnt64, [vv], ones8, mask=lt8)

            pltpu.sync_copy(oi.at[pl.ds(0, CH * TOP_K)],
                            idx_hbm.at[pl.ds(tok0 * TOP_K, CH * TOP_K)])
            pltpu.sync_copy(ow.at[pl.ds(0, CH * TOP_K)],
                            w_hbm.at[pl.ds(tok0 * TOP_K, CH * TOP_K)])

        pltpu.sync_copy(cnt64, cnt_hbm.at[pl.ds(wid * N_EXPERTS, N_EXPERTS)])

    return _sc_topk_body


def _sc_call(logits, tokens):
    tw = tokens // NW
    mesh = plsc.VectorSubcoreMesh(core_axis_name="c", subcore_axis_name="s")
    f = functools.partial(
        pl.kernel, mesh=mesh,
        out_type=[
            jax.ShapeDtypeStruct((tokens * TOP_K,), jnp.int32),
            jax.ShapeDtypeStruct((tokens * TOP_K,), jnp.float32),
            jax.ShapeDtypeStruct((NW * N_EXPERTS,), jnp.float32),
        ],
        scratch_types=[
            pltpu.VMEM((CH * LROW,), jnp.float32),
            pltpu.VMEM((CH * TOP_K + 8,), jnp.int32),
            pltpu.VMEM((CH * TOP_K + 8,), jnp.float32),
            pltpu.VMEM((N_EXPERTS,), jnp.float32),
        ],
        compiler_params=pltpu.CompilerParams(needs_layout_passes=False),
    )(_make_sc_body(tw))
    return f(logits.reshape(-1))


def _loss_body(*refs):
    cnt_refs = refs[:NSPLIT]
    s_refs = refs[NSPLIT:2 * NSPLIT]
    out_ref = refs[2 * NSPLIT]
    cnt = jnp.zeros((N_EXPERTS,), jnp.float32)
    for r in cnt_refs:
        for j in range(NW):
            cnt = cnt + r[pl.ds(j * N_EXPERTS, N_EXPERTS)]
    s = s_refs[0][...]
    for r in s_refs[1:]:
        s = s + r[...]
    aux = jnp.sum(cnt * s) * (AUX_ALPHA / (TOKENS * float(TOKENS)))
    z = jnp.mean(jnp.log(s) ** 2) * Z_ALPHA
    out_ref[...] = jnp.broadcast_to(aux + z, (1, 1))


def _loss_call(cnts, ss):
    return pl.pallas_call(
        _loss_body,
        out_shape=jax.ShapeDtypeStruct((1, 1), jnp.float32),
    )(*cnts, *ss)


def kernel(hidden_states, gate_weight):
    wt = gate_weight
    tiles = TOKENS // TILE_T
    per = tiles // NSPLIT
    gates = [_gate_call(hidden_states, wt, j * per, per)
             for j in range(NSPLIT)]
    scs = [_sc_call(lg, per * TILE_T) for (lg, _) in gates]
    topk_idx = jnp.concatenate([o[0] for o in scs]).reshape(TOKENS, TOP_K)
    topk_weight = jnp.concatenate([o[1] for o in scs]).reshape(TOKENS, TOP_K)
    loss = _loss_call([o[2] for o in scs], [g[1] for g in gates])
    return (topk_idx, topk_weight, loss[0, 0])
